# baseline (speedup 1.0000x reference)
import functools

import jax
import jax.numpy as jnp
from jax import lax
from jax.experimental import pallas as pl
from jax.experimental.pallas import tpu as pltpu
from jax.experimental.pallas import tpu_sc as plsc

HIDDEN = 2048
NC, NS = 2, 16
NW = NC * NS
C = 16
EPS = 1e-12
SEQ_BLK = 256


def _make_gather(n_tokens):
    rows_pw = n_tokens // NW
    n_chunks = rows_pw // C

    @functools.partial(
        pl.kernel,
        out_type=jax.ShapeDtypeStruct((n_tokens, HIDDEN), jnp.float32),
        mesh=plsc.VectorSubcoreMesh(core_axis_name="c", subcore_axis_name="s"),
        compiler_params=pltpu.CompilerParams(needs_layout_passes=False),
        scratch_types=[
            pltpu.VMEM((n_chunks, C), jnp.int32),
            pltpu.VMEM((C, HIDDEN), jnp.float32),
            pltpu.VMEM((C, HIDDEN), jnp.float32),
            pltpu.VMEM((C, HIDDEN), jnp.float32),
            pltpu.SemaphoreType.DMA,
            pltpu.SemaphoreType.DMA,
            pltpu.SemaphoreType.DMA,
            pltpu.SemaphoreType.DMA,
            pltpu.SemaphoreType.DMA,
            pltpu.SemaphoreType.DMA,
        ],
    )
    def gather(ids_hbm, tok_hbm, out_hbm,
               ids_v, buf0, buf1, buf2, sg0, sg1, sg2, so0, so1, so2):
        wid = lax.axis_index("s") * NC + lax.axis_index("c")
        row_base = wid * rows_pw
        pltpu.sync_copy(ids_hbm.at[wid], ids_v)

        bufs = (buf0, buf1, buf2)
        gsems = (sg0, sg1, sg2)
        osems = (so0, so1, so2)
        nbuf = 3

        def start_gather(j, b):
            return pltpu.async_copy(tok_hbm.at[ids_v.at[j]], bufs[b],
                                    gsems[b])

        def start_out(j, b):
            return pltpu.async_copy(
                bufs[b], out_hbm.at[pl.ds(row_base + j * C, C)], osems[b])

        def wait_gather(j, b):
            pltpu.make_async_copy(tok_hbm.at[ids_v.at[j]], bufs[b],
                                  gsems[b]).wait()

        def wait_out(j, b):
            pltpu.make_async_copy(
                bufs[b], out_hbm.at[pl.ds(row_base + j * C, C)],
                osems[b]).wait()

        def process(j, b, issue_next):
            wait_gather(j, b)
            start_out(j, b)
            wait_out(j, b)
            if issue_next:
                start_gather(j + nbuf, b)

        for b in range(nbuf):
            start_gather(b, b)

        n_loop = (n_chunks - nbuf) // nbuf

        def body(m, _):
            for b in range(nbuf):
                process(m * nbuf + b, b, True)
            return 0

        lax.fori_loop(0, n_loop, body, 0)

        for j in range(n_loop * nbuf, n_chunks):
            process(j, j % nbuf, j + nbuf < n_chunks)

    return gather


def _ln_body(x_ref, pos_ref, g_ref, b_ref, o_ref):
    x = x_ref[...] + pos_ref[...][None, :, :]
    mean = jnp.mean(x, axis=-1, keepdims=True)
    xc = x - mean
    var = jnp.mean(xc * xc, axis=-1, keepdims=True)
    o_ref[...] = (xc * lax.rsqrt(var + EPS) * g_ref[...][None, :, :]
                  + b_ref[...][None, :, :])


def _ln(x, pos_table, g, b, B, S):
    grid = (S // SEQ_BLK,)
    return pl.pallas_call(
        _ln_body,
        grid=grid,
        in_specs=[
            pl.BlockSpec((B, SEQ_BLK, HIDDEN), lambda i: (0, i, 0)),
            pl.BlockSpec((SEQ_BLK, HIDDEN), lambda i: (i, 0)),
            pl.BlockSpec((1, HIDDEN), lambda i: (0, 0)),
            pl.BlockSpec((1, HIDDEN), lambda i: (0, 0)),
        ],
        out_specs=pl.BlockSpec((B, SEQ_BLK, HIDDEN), lambda i: (0, i, 0)),
        out_shape=jax.ShapeDtypeStruct((B, S, HIDDEN), jnp.float32),
    )(x, pos_table, g.reshape(1, HIDDEN), b.reshape(1, HIDDEN))


def kernel(input_ids, token_table, pos_table, ln_gamma, ln_beta):
    B, S = input_ids.shape
    n = B * S
    ids = input_ids.reshape(NW, (n // NW) // C, C).astype(jnp.int32)
    gathered = _make_gather(n)(ids, token_table)
    return _ln(gathered.reshape(B, S, HIDDEN), pos_table,
               ln_gamma.astype(jnp.float32), ln_beta.astype(jnp.float32),
               B, S)
